# baseline (speedup 1.0000x reference)
import functools

import jax
import jax.numpy as jnp
from jax import lax
from jax.experimental import pallas as pl
from jax.experimental.pallas import tpu as pltpu
from jax.experimental.pallas import tpu_sc as plsc

_INFO = plsc.get_sparse_core_info()
_NC = _INFO.num_cores
_NS = _INFO.num_subcores
_NW = _NC * _NS
_L = 16


def _sc_loss_kernel(pred_lin, pk, B, C, H, W, K, NT):
    HW = H * W
    KG = NT // _L
    mesh = plsc.VectorSubcoreMesh(core_axis_name="c", subcore_axis_name="s")

    @functools.partial(
        pl.kernel,
        out_type=jax.ShapeDtypeStruct((_NW, 2 * _L), jnp.float32),
        mesh=mesh,
        scratch_types=[
            pltpu.VMEM((5 * NT,), jnp.float32),
            pltpu.VMEM((4 * NT,), jnp.int32),
            pltpu.VMEM((4 * NT,), jnp.float32),
            pltpu.VMEM((2 * _L,), jnp.float32),
            pltpu.SemaphoreType.DMA,
        ],
    )
    def body(pred_hbm, pk_hbm, out_hbm, pk_v, idx_v, gv_v, out_v, gsem):
        wid = lax.axis_index("s") * _NC + lax.axis_index("c")

        pltpu.sync_copy(pk_hbm.at[wid], pk_v)

        base = wid * (C * HW)

        def build_quarter(q, _):
            j, kind, u = q >> 2, (q >> 1) & 1, q & 1
            cbase = j * 128
            p = pk_v[pl.ds(kind * NT + j * 2 * _L + u * _L, _L)
                     ].astype(jnp.int32)
            h = p >> 8
            w = p & (W - 1)
            tiled = (((h >> 3) << 11) + ((w >> 7) << 10)
                     + ((h & 7) << 7) + (w & 127))
            a0 = base + tiled
            koff = cbase + kind * 64 + u * _L
            idx_v[pl.ds(koff, _L)] = a0
            idx_v[pl.ds(koff + 2 * _L, _L)] = a0 + HW

            @pl.when(q & 3 == 3)
            def _fire():
                sl = pl.ds(cbase, 128)
                pltpu.async_copy(pred_hbm.at[idx_v.at[sl]], gv_v.at[sl],
                                 gsem)
            return _

        lax.fori_loop(0, 2 * KG, build_quarter, None)
        pltpu.make_async_copy(pred_hbm.at[pl.ds(0, 4 * NT)], gv_v,
                              gsem).wait()

        def accum(i, carry):
            acc, mac = carry
            g, c = i >> 1, i & 1
            cbase = (g >> 1) * 128
            u = (g & 1) * _L
            m = pk_v[pl.ds(4 * NT + g * _L, _L)]
            vs = gv_v[pl.ds(cbase + c * 2 * _L + u, _L)]
            ve = gv_v[pl.ds(cbase + 64 + c * 2 * _L + u, _L)]
            t = pk_v[pl.ds((2 + c) * NT + g * _L, _L)]
            gavg = (vs + ve) * 0.5
            d = gavg * m - t * m
            ad = jnp.abs(d)
            l = jnp.where(ad < 1.0, 0.5 * d * d, ad - 0.5)
            return acc + l, mac + m

        acc, mac = lax.fori_loop(
            0, 2 * KG, accum,
            (jnp.zeros((_L,), jnp.float32), jnp.zeros((_L,), jnp.float32)))
        out_v[pl.ds(0, _L)] = acc
        out_v[pl.ds(_L, _L)] = mac
        pltpu.sync_copy(out_v, out_hbm.at[wid])

    return body(pred_lin, pk)


def kernel(pred, mask, ind, target):
    B, C, H, W = pred.shape
    K = ind.shape[1]
    NT = ((K + _L - 1) // _L + 7) // 8 * 8 * _L
    pk = NT - K

    pred_lin = (pred.reshape(B, C, H // 8, 8, W // 128, 128)
                .swapaxes(3, 4).reshape(-1))

    pk_op = jnp.concatenate(
        [jnp.pad(ind.astype(jnp.float32).swapaxes(1, 2),
                 ((0, 0), (0, 0), (0, pk))),
         jnp.pad(target.swapaxes(1, 2), ((0, 0), (0, 0), (0, pk))),
         jnp.pad(mask, ((0, 0), (0, pk)))[:, None, :]],
        axis=1).reshape(B, 5 * NT)

    out = _sc_loss_kernel(pred_lin, pk_op, B, C, H, W, K, NT)
    s = jnp.sum(out, axis=0)
    num = jnp.sum(s[:_L])
    den = jnp.sum(s[_L:])
    return num / (den + 0.0001)


# speedup vs baseline: 1.0034x; 1.0034x over previous
import functools

import jax
import jax.numpy as jnp
from jax import lax
from jax.experimental import pallas as pl
from jax.experimental.pallas import tpu as pltpu
from jax.experimental.pallas import tpu_sc as plsc

_INFO = plsc.get_sparse_core_info()
_NC = _INFO.num_cores
_NS = _INFO.num_subcores
_NW = _NC * _NS
_L = 16


def _sc_loss_kernel(pred_lin, pk, B, C, H, W, K, NT):
    HW = H * W
    KG = NT // _L
    mesh = plsc.VectorSubcoreMesh(core_axis_name="c", subcore_axis_name="s")

    @functools.partial(
        pl.kernel,
        out_type=jax.ShapeDtypeStruct((_NW, 2 * _L), jnp.float32),
        mesh=mesh,
        scratch_types=[
            pltpu.VMEM((5 * NT,), jnp.float32),
            pltpu.VMEM((4 * NT,), jnp.int32),
            pltpu.VMEM((4 * NT,), jnp.float32),
            pltpu.VMEM((2 * _L,), jnp.float32),
            pltpu.SemaphoreType.DMA,
        ],
    )
    def body(pred_hbm, pk_hbm, out_hbm, pk_v, idx_v, gv_v, out_v, gsem):
        wid = lax.axis_index("s") * _NC + lax.axis_index("c")

        pltpu.sync_copy(pk_hbm.at[wid], pk_v)

        base = wid * (C * HW)

        def build_chunk(j, _):
            cbase = j * 128
            for kind in (0, 1):
                for u in (0, 1):
                    p = pk_v[pl.ds(kind * NT + cbase // 4 + u * _L, _L)
                             ].astype(jnp.int32)
                    h = p >> 8
                    w = p & (W - 1)
                    tiled = (((h >> 3) << 11) + ((w >> 7) << 10)
                             + ((h & 7) << 7) + (w & 127))
                    a0 = base + tiled
                    koff = cbase + kind * 64 + u * _L
                    idx_v[pl.ds(koff, _L)] = a0
                    idx_v[pl.ds(koff + 2 * _L, _L)] = a0 + HW
            sl = pl.ds(cbase, 128)
            pltpu.async_copy(pred_hbm.at[idx_v.at[sl]], gv_v.at[sl], gsem)
            return _

        lax.fori_loop(0, KG // 2, build_chunk, None)
        pltpu.make_async_copy(pred_hbm.at[pl.ds(0, 4 * NT)], gv_v,
                              gsem).wait()

        def accum(g, carry):
            acc, mac = carry
            cbase = (g >> 1) * 128
            u = (g & 1) * _L
            m = pk_v[pl.ds(4 * NT + g * _L, _L)]
            for c in (0, 1):
                vs = gv_v[pl.ds(cbase + c * 2 * _L + u, _L)]
                ve = gv_v[pl.ds(cbase + 64 + c * 2 * _L + u, _L)]
                t = pk_v[pl.ds((2 + c) * NT + g * _L, _L)]
                gavg = (vs + ve) * 0.5
                d = gavg * m - t * m
                ad = jnp.abs(d)
                l = jnp.where(ad < 1.0, 0.5 * d * d, ad - 0.5)
                acc = acc + l
                mac = mac + m
            return acc, mac

        acc, mac = lax.fori_loop(
            0, KG, accum,
            (jnp.zeros((_L,), jnp.float32), jnp.zeros((_L,), jnp.float32)))
        out_v[pl.ds(0, _L)] = acc
        out_v[pl.ds(_L, _L)] = mac
        pltpu.sync_copy(out_v, out_hbm.at[wid])

    return body(pred_lin, pk)


def kernel(pred, mask, ind, target):
    B, C, H, W = pred.shape
    K = ind.shape[1]
    NT = ((K + _L - 1) // _L + 7) // 8 * 8 * _L
    pk = NT - K

    pred_lin = (pred.reshape(B, C, H // 8, 8, W // 128, 128)
                .swapaxes(3, 4).reshape(-1))

    pk_op = jnp.concatenate(
        [jnp.pad(ind.astype(jnp.float32).swapaxes(1, 2),
                 ((0, 0), (0, 0), (0, pk))),
         jnp.pad(target.swapaxes(1, 2), ((0, 0), (0, 0), (0, pk))),
         jnp.pad(mask, ((0, 0), (0, pk)))[:, None, :]],
        axis=1).reshape(B, 5 * NT)

    out = _sc_loss_kernel(pred_lin, pk_op, B, C, H, W, K, NT)
    out = out.reshape(_NW, 2, _L)
    num = jnp.sum(out[:, 0, :])
    den = jnp.sum(out[:, 1, :])
    return num / (den + 0.0001)
